# baseline (speedup 1.0000x reference)
import functools

import jax
import jax.numpy as jnp
from jax import lax
from jax.experimental import pallas as pl
from jax.experimental.pallas import tpu as pltpu
from jax.experimental.pallas import tpu_sc as plsc

N = 20000
G = 100
C = 80
L = 128
BLK = 2000
NB = N // BLK
BLKD = 2048
NBD = 10
BIG_F = 1.0e30
SENT_F = 3.0e38
BIG_I = 2 ** 30

ALPHA = 0.25
COST_CLASS = 2.0
COST_BBOX = 5.0
COST_GIOU = 2.0

_pallas_call = pl.pallas_call


def _build_cost(boxes_ref, poses_ref, plog, pk1_ref, pk2_ref):
    x0 = boxes_ref[:, 0:1]
    y0 = boxes_ref[:, 1:2]
    x1 = boxes_ref[:, 2:3]
    y1 = boxes_ref[:, 3:4]

    X0 = pk1_ref[0:1, :]
    Y0 = pk1_ref[1:2, :]
    X1 = pk1_ref[2:3, :]
    Y1 = pk1_ref[3:4, :]
    TNX0 = pk1_ref[4:5, :]
    TNY0 = pk1_ref[5:6, :]
    TNX1 = pk1_ref[6:7, :]
    TNY1 = pk1_ref[7:8, :]
    TT0 = pk1_ref[8:9, :]
    TT1 = pk1_ref[9:10, :]
    TT2 = pk1_ref[10:11, :]
    TR0 = pk1_ref[11:12, :]
    TR1 = pk1_ref[12:13, :]
    TR2 = pk1_ref[13:14, :]
    AREA2 = pk1_ref[14:15, :]

    BX0 = pk2_ref[0:1, :]
    BY0 = pk2_ref[1:2, :]
    BX1 = pk2_ref[2:3, :]
    BY1 = pk2_ref[3:4, :]
    CLo = pk2_ref[4:5, :]
    CHi = pk2_ref[5:6, :]
    CTo = pk2_ref[6:7, :]
    CBo = pk2_ref[7:8, :]

    area1 = (x1 - x0) * (y1 - y0)
    ltx = jnp.maximum(x0, X0)
    lty = jnp.maximum(y0, Y0)
    rbx = jnp.minimum(x1, X1)
    rby = jnp.minimum(y1, Y1)
    iw = jnp.clip(rbx - ltx, 0.0, None)
    ih = jnp.clip(rby - lty, 0.0, None)
    inter = iw * ih
    union = area1 + AREA2 - inter
    iou = inter / union
    ex = jnp.minimum(x0, X0)
    exr = jnp.maximum(x1, X1)
    ey = jnp.minimum(y0, Y0)
    eyb = jnp.maximum(y1, Y1)
    earea = jnp.clip(exr - ex, 0.0, None) * jnp.clip(eyb - ey, 0.0, None)
    giou = iou - (earea - union) / earea

    p = 1.0 / (1.0 + jnp.exp(-plog))
    one_m_p = 1.0 - p
    pos = ALPHA * one_m_p * one_m_p * (-jnp.log(p + 1e-8))
    neg = (1.0 - ALPHA) * p * p * (-jnp.log(1.0 - p + 1e-8))
    cost_class = pos - neg

    inv_w = jnp.float32(1.0) / jnp.float32(1333.0)
    inv_h = jnp.float32(1.0) / jnp.float32(800.0)
    cb = (jnp.abs(x0 * inv_w - TNX0) + jnp.abs(y0 * inv_h - TNY0)
          + jnp.abs(x1 * inv_w - TNX1) + jnp.abs(y1 * inv_h - TNY1))

    t0 = poses_ref[:, 0:1]
    t1 = poses_ref[:, 1:2]
    t2 = poses_ref[:, 2:3]
    r0 = poses_ref[:, 3:4]
    r1 = poses_ref[:, 4:5]
    r2 = poses_ref[:, 5:6]
    cpose = (jnp.abs(t0 - TT0) + jnp.abs(t1 - TT1) + jnp.abs(t2 - TT2)
             + jnp.abs(r0 - TR0) + jnp.abs(r1 - TR1) + jnp.abs(r2 - TR2))

    ax = (x0 + x1) * 0.5
    ay = (y0 + y1) * 0.5
    in_boxes = ((ax > BX0) & (ax < BX1) & (ay > BY0) & (ay < BY1))
    in_centers = ((ax > CLo) & (ax < CHi) & (ay > CTo) & (ay < CBo))
    both = in_boxes & in_centers
    fg = (jnp.sum(in_boxes.astype(jnp.float32), axis=1, keepdims=True) > 0.0) | \
         (jnp.sum(in_centers.astype(jnp.float32), axis=1, keepdims=True) > 0.0)

    cost = (COST_BBOX * cb + COST_CLASS * cost_class + COST_GIOU * (-giou)
            + 100.0 * (1.0 - both.astype(jnp.float32)) + cpose
            + 10000.0 * (1.0 - fg.astype(jnp.float32)))

    lane = lax.broadcasted_iota(jnp.int32, (BLK, L), 1)
    cost = jnp.where(lane < G, cost, BIG_F)
    iou = jnp.where(lane < G, iou, -1.0)
    return cost, iou


def _extract5_min(cur):
    ms = []
    for t in range(5):
        m = jnp.min(cur, axis=0, keepdims=True)
        ms.append(m)
        if t < 4:
            cur = jnp.where(cur <= m, SENT_F, cur)
    return jnp.concatenate(ms, axis=0)


def _extract5_max0(cur):
    ms = []
    for t in range(5):
        m = jnp.max(cur, axis=0, keepdims=True)
        ms.append(jnp.maximum(m, 0.0))
        if t < 4:
            cur = jnp.where(cur >= m, -SENT_F, cur)
    return jnp.concatenate(ms, axis=0)


def _threshold(s_cost, s_iou):
    s = (s_iou[0:1, :] + s_iou[1:2, :] + s_iou[2:3, :]
         + s_iou[3:4, :] + s_iou[4:5, :])
    t = s_cost[0:1, :]
    t = jnp.where(s >= 2.0, s_cost[1:2, :], t)
    t = jnp.where(s >= 3.0, s_cost[2:3, :], t)
    t = jnp.where(s >= 4.0, s_cost[3:4, :], t)
    t = jnp.where(s >= 5.0, s_cost[4:5, :], t)
    return t


def _ka(boxes_ref, poses_ref, lt_ref, oh_ref, pk1_ref, pk2_ref,
        sel_out, gt_out, qidx_out, reff_out,
        costS, s_cost, s_iou, acc_colsum, acc_bval, acc_bidx,
        acc_mval, acc_midx):
    pid = pl.program_id(0)

    @pl.when(pid == 0)
    def _init():
        s_cost[...] = jnp.full((8, L), SENT_F, jnp.float32)
        s_iou[...] = jnp.full((8, L), -SENT_F, jnp.float32)
        acc_colsum[...] = jnp.zeros((8, L), jnp.float32)
        acc_bval[...] = jnp.full((8, L), SENT_F, jnp.float32)
        acc_bidx[...] = jnp.zeros((8, L), jnp.int32)
        acc_mval[...] = jnp.full((8, L), SENT_F, jnp.float32)
        acc_midx[...] = jnp.zeros((8, L), jnp.int32)

    @pl.when(pid < NBD)
    def _phase0():
        d = lax.dot_general(lt_ref[...], oh_ref[...],
                            (((0,), (0,)), ((), ())),
                            precision=lax.Precision.HIGHEST,
                            preferred_element_type=jnp.float32)

        @pl.when(pid < NBD - 1)
        def _full():
            costS[pl.ds(pid * BLKD, BLKD), :] = d

        @pl.when(pid == NBD - 1)
        def _tail():
            costS[pl.ds((NBD - 1) * BLKD, N - (NBD - 1) * BLKD), :] = \
                d[0:N - (NBD - 1) * BLKD, :]

    @pl.when((pid >= NBD) & (pid < NBD + NB))
    def _phase1():
        b = pid - NBD
        cost, iou = _build_cost(boxes_ref, poses_ref,
                                costS[pl.ds(b * BLK, BLK), :],
                                pk1_ref, pk2_ref)
        costS[pl.ds(b * BLK, BLK), :] = cost
        blk5 = _extract5_min(cost)
        s_cost[0:5, :] = _extract5_min(
            jnp.concatenate([blk5, s_cost[0:5, :]], axis=0))
        blk5i = _extract5_max0(iou)
        s_iou[0:5, :] = _extract5_max0(
            jnp.concatenate([blk5i, s_iou[0:5, :]], axis=0))

    @pl.when((pid >= NBD + NB) & (pid < NBD + 2 * NB))
    def _phase2():
        b = pid - (NBD + NB)
        cost = costS[pl.ds(b * BLK, BLK), :]
        t = _threshold(s_cost, s_iou)

        lane = lax.broadcasted_iota(jnp.int32, (BLK, L), 1)
        valid = lane < G
        matching0 = (cost <= t) & valid
        amg = jnp.sum(matching0.astype(jnp.float32), axis=1, keepdims=True)

        rmin = jnp.min(cost, axis=1, keepdims=True)
        amin = jnp.min(jnp.where(cost == rmin, lane, BIG_I),
                       axis=1, keepdims=True)
        onehot_f = (lane == amin).astype(jnp.float32)
        m0f = matching0.astype(jnp.float32)
        conflict_f = (amg > 1.0).astype(jnp.float32)
        mf = conflict_f * onehot_f + (1.0 - conflict_f) * m0f

        matched = amg > 0.0
        matched_f = matched.astype(jnp.float32)
        firstlane = jnp.min(jnp.where(mf > 0.0, lane, BIG_I),
                            axis=1, keepdims=True)
        gt = jnp.where(matched, firstlane, 0)

        sel_out[...] = jnp.broadcast_to(matched_f, (BLK, 8))
        gt_out[...] = jnp.broadcast_to(gt, (BLK, 8))

        acc_colsum[0:1, :] = acc_colsum[0:1, :] + \
            jnp.sum(mf, axis=0, keepdims=True)

        rowg = lax.broadcasted_iota(jnp.int32, (BLK, L), 0) + b * BLK
        boosted = cost + 100000.0 * matched_f
        bval = jnp.min(boosted, axis=0, keepdims=True)
        bidx = jnp.min(jnp.where(boosted == bval, rowg, BIG_I),
                       axis=0, keepdims=True)
        old_v = acc_bval[0:1, :]
        old_i = acc_bidx[0:1, :]
        upd = bval < old_v
        acc_bval[0:1, :] = jnp.where(upd, bval, old_v)
        acc_bidx[0:1, :] = jnp.where(upd, bidx, old_i)

        mrow = jnp.where(matched, cost, SENT_F)
        mval = jnp.min(mrow, axis=0, keepdims=True)
        midx = jnp.min(jnp.where(mrow == mval, rowg, BIG_I),
                       axis=0, keepdims=True)
        old_v = acc_mval[0:1, :]
        old_i = acc_midx[0:1, :]
        upd = mval < old_v
        acc_mval[0:1, :] = jnp.where(upd, mval, old_v)
        acc_midx[0:1, :] = jnp.where(upd, midx, old_i)

    @pl.when(pid == NBD + 2 * NB)
    def _phase3():
        lane1 = lax.broadcasted_iota(jnp.int32, (1, L), 1)
        active = (acc_colsum[0:1, :] == 0.0) & (lane1 < G)
        r_row = jnp.where(active, acc_bidx[0:1, :], N)

        def body(i, carry):
            rv, ri = carry
            c = costS[pl.ds(i * BLK, BLK), :]
            rowg = lax.broadcasted_iota(jnp.int32, (BLK, L), 0) + i * BLK
            eq = rowg == r_row
            member = jnp.sum(eq.astype(jnp.float32), axis=1,
                             keepdims=True) > 0.0
            vals = jnp.where(member, c, SENT_F)
            v = jnp.min(vals, axis=0, keepdims=True)
            idx = jnp.min(jnp.where(vals == v, rowg, BIG_I),
                          axis=0, keepdims=True)
            upd = v < rv
            return (jnp.where(upd, v, rv), jnp.where(upd, idx, ri))

        rv0 = jnp.full((1, L), SENT_F, jnp.float32)
        ri0 = jnp.zeros((1, L), jnp.int32)
        rv, ri = lax.fori_loop(0, NB, body, (rv0, ri0))

        mv = acc_mval[0:1, :]
        mi = acc_midx[0:1, :]
        q = jnp.where(rv < mv, ri, mi)
        q = jnp.where(rv == mv, jnp.minimum(ri, mi), q)
        qidx_out[...] = jnp.broadcast_to(q, (8, L))

        io0 = lax.broadcasted_iota(jnp.int32, (L, L), 0)
        io1 = lax.broadcasted_iota(jnp.int32, (L, L), 1)
        ident = io0 == io1
        Rb = jnp.broadcast_to(r_row, (L, L))
        rT = jnp.min(jnp.where(ident, Rb, BIG_I), axis=1, keepdims=True)
        act_i = active.astype(jnp.int32)
        Ab = jnp.broadcast_to(act_i, (L, L))
        aT = jnp.min(jnp.where(ident, Ab, BIG_I), axis=1, keepdims=True)
        m2 = (rT == r_row) & (aT == 1)
        gval = jnp.min(jnp.where(m2, io0, BIG_I), axis=0, keepdims=True)

        idx_eff = jnp.where(active, 8 * acc_bidx[0:1, :], 1)
        reff_out[...] = jnp.concatenate(
            [r_row, idx_eff, gval,
             jnp.zeros((5, L), jnp.int32)], axis=0)


def _kb_sc(sel_in, gt_in, reff_in, sel_o, gt_o, idxv, gvalv, onesv, sem):
    c = lax.axis_index("c")
    s = lax.axis_index("s")

    @pl.when((c == 0) & (s == 0))
    def _():
        pltpu.sync_copy(sel_in, sel_o)
        pltpu.sync_copy(gt_in, gt_o)
        pltpu.sync_copy(reff_in.at[1], idxv)
        pltpu.sync_copy(reff_in.at[2], gvalv)
        for k in range(8):
            onesv[pl.ds(k * 16, 16)] = jnp.full((16,), 1.0, jnp.float32)
        pltpu.async_copy(onesv, sel_o.at[idxv], sem).wait()
        pltpu.async_copy(gvalv, gt_o.at[idxv], sem).wait()


def _k3tc(sel0_ref, gt0_ref, reff_ref, sel_out, gtf_out):
    pid = pl.program_id(0)
    r_row = reff_ref[0:1, :]
    rowg = lax.broadcasted_iota(jnp.int32, (BLK, L), 0) + pid * BLK
    eq = rowg == r_row
    anyeq = jnp.sum(eq.astype(jnp.float32), axis=1, keepdims=True) > 0.0
    lane = lax.broadcasted_iota(jnp.int32, (BLK, L), 1)
    gmin = jnp.min(jnp.where(eq, lane, BIG_I), axis=1, keepdims=True)

    matched = sel0_ref[:, 0:1] > 0.0
    sel = matched | anyeq
    gt = jnp.where((~matched) & anyeq, gmin, gt0_ref[:, 0:1])
    sel_out[...] = jnp.broadcast_to(sel.astype(jnp.float32), (BLK, 8))
    gtf_out[...] = jnp.broadcast_to(gt, (BLK, 8))


def kernel(pred_logits, pred_boxes, pred_poses, tgt_labels, tgt_boxes,
           tgt_boxes_xyxy, P2s, image_size_xyxy, image_size_xyxy_tgt,
           translation_matrix, rotation_matrix, lwhs):
    boxes = pred_boxes[0]
    poses = pred_poses[0]
    logitsT = jnp.transpose(pred_logits[0])

    onehot = (tgt_labels[None, :] ==
              jnp.arange(C, dtype=tgt_labels.dtype)[:, None])
    onehot = jnp.pad(onehot.astype(jnp.float32), ((0, 0), (0, L - G)))

    tx0, ty0 = tgt_boxes_xyxy[:, 0], tgt_boxes_xyxy[:, 1]
    tx1, ty1 = tgt_boxes_xyxy[:, 2], tgt_boxes_xyxy[:, 3]
    tnorm = tgt_boxes_xyxy / image_size_xyxy_tgt
    area2 = (tx1 - tx0) * (ty1 - ty0)
    pk1 = jnp.stack([tx0, ty0, tx1, ty1,
                     tnorm[:, 0], tnorm[:, 1], tnorm[:, 2], tnorm[:, 3],
                     translation_matrix[:, 0], translation_matrix[:, 1],
                     translation_matrix[:, 2],
                     rotation_matrix[:, 0], rotation_matrix[:, 1],
                     rotation_matrix[:, 2], area2,
                     jnp.zeros_like(tx0)], axis=0)
    pk1 = jnp.pad(pk1, ((0, 0), (0, L - G)))

    tcx, tcy = (tx0 + tx1) * 0.5, (ty0 + ty1) * 0.5
    tw, th = tx1 - tx0, ty1 - ty0
    BX0, BY0 = tcx - 0.5 * tw, tcy - 0.5 * th
    BX1, BY1 = tcx + 0.5 * tw, tcy + 0.5 * th
    cr = 2.5
    CLo = tcx - cr * (BX1 - BX0)
    CHi = tcx + cr * (BX1 - BX0)
    CTo = tcy - cr * (BY1 - BY0)
    CBo = tcy + cr * (BY1 - BY0)
    pk2 = jnp.stack([BX0, BY0, BX1, BY1, CLo, CHi, CTo, CBo], axis=0)
    pk2 = jnp.pad(pk2, ((0, 0), (0, L - G)), constant_values=BIG_F)

    def in_map(i):
        return (jnp.clip(i - NBD, 0, NB - 1), 0)

    def out_map(i):
        return (jnp.clip(i - (NBD + NB), 0, NB - 1), 0)

    sel0, gt0, qidx, reff = _pallas_call(
        _ka,
        grid=(NBD + 2 * NB + 1,),
        in_specs=[
            pl.BlockSpec((BLK, 4), in_map),
            pl.BlockSpec((BLK, 6), in_map),
            pl.BlockSpec((C, BLKD), lambda i: (0, jnp.minimum(i, NBD - 1))),
            pl.BlockSpec((C, L), lambda i: (0, 0)),
            pl.BlockSpec((16, L), lambda i: (0, 0)),
            pl.BlockSpec((8, L), lambda i: (0, 0)),
        ],
        out_specs=[
            pl.BlockSpec((BLK, 8), out_map),
            pl.BlockSpec((BLK, 8), out_map),
            pl.BlockSpec((8, L), lambda i: (0, 0)),
            pl.BlockSpec((8, L), lambda i: (0, 0)),
        ],
        out_shape=[
            jax.ShapeDtypeStruct((N, 8), jnp.float32),
            jax.ShapeDtypeStruct((N, 8), jnp.int32),
            jax.ShapeDtypeStruct((8, L), jnp.int32),
            jax.ShapeDtypeStruct((8, L), jnp.int32),
        ],
        scratch_shapes=[
            pltpu.VMEM((N, L), jnp.float32),
            pltpu.VMEM((8, L), jnp.float32),
            pltpu.VMEM((8, L), jnp.float32),
            pltpu.VMEM((8, L), jnp.float32),
            pltpu.VMEM((8, L), jnp.float32),
            pltpu.VMEM((8, L), jnp.int32),
            pltpu.VMEM((8, L), jnp.float32),
            pltpu.VMEM((8, L), jnp.int32),
        ],
    )(boxes, poses, logitsT, onehot, pk1, pk2)

    kb = functools.partial(
        pl.kernel,
        mesh=plsc.VectorSubcoreMesh(core_axis_name="c", subcore_axis_name="s"),
        out_type=[
            jax.ShapeDtypeStruct((N * 8,), jnp.float32),
            jax.ShapeDtypeStruct((N * 8,), jnp.int32),
        ],
        scratch_types=[
            pltpu.VMEM((L,), jnp.int32),
            pltpu.VMEM((L,), jnp.int32),
            pltpu.VMEM((L,), jnp.float32),
            pltpu.SemaphoreType.DMA,
        ],
    )(_kb_sc)
    sel, gtf = kb(sel0.reshape(-1), gt0.reshape(-1), reff)

    selected_query = sel.reshape(N, 8)[:, 0] > 0.0
    gt_indices = gtf.reshape(N, 8)[:, 0]
    matched_query_id = qidx[0, :G]
    return selected_query, gt_indices, matched_query_id


# speedup vs baseline: 1.1512x; 1.1512x over previous
import functools

import jax
import jax.numpy as jnp
from jax import lax
from jax.experimental import pallas as pl
from jax.experimental.pallas import tpu as pltpu
from jax.experimental.pallas import tpu_sc as plsc

N = 20000
G = 100
C = 80
L = 128
BLK = 2000
NB = N // BLK
BLKD = 2048
NBD = 10
BIG_F = 1.0e30
SENT_F = 3.0e38
BIG_I = 2 ** 30

ALPHA = 0.25
COST_CLASS = 2.0
COST_BBOX = 5.0
COST_GIOU = 2.0

_pallas_call = pl.pallas_call


def _build_cost(boxes_ref, poses_ref, plog, pk1_ref, pk2_ref):
    x0 = boxes_ref[:, 0:1]
    y0 = boxes_ref[:, 1:2]
    x1 = boxes_ref[:, 2:3]
    y1 = boxes_ref[:, 3:4]

    X0 = pk1_ref[0:1, :]
    Y0 = pk1_ref[1:2, :]
    X1 = pk1_ref[2:3, :]
    Y1 = pk1_ref[3:4, :]
    TNX0 = pk1_ref[4:5, :]
    TNY0 = pk1_ref[5:6, :]
    TNX1 = pk1_ref[6:7, :]
    TNY1 = pk1_ref[7:8, :]
    TT0 = pk1_ref[8:9, :]
    TT1 = pk1_ref[9:10, :]
    TT2 = pk1_ref[10:11, :]
    TR0 = pk1_ref[11:12, :]
    TR1 = pk1_ref[12:13, :]
    TR2 = pk1_ref[13:14, :]
    AREA2 = pk1_ref[14:15, :]

    BX0 = pk2_ref[0:1, :]
    BY0 = pk2_ref[1:2, :]
    BX1 = pk2_ref[2:3, :]
    BY1 = pk2_ref[3:4, :]
    CLo = pk2_ref[4:5, :]
    CHi = pk2_ref[5:6, :]
    CTo = pk2_ref[6:7, :]
    CBo = pk2_ref[7:8, :]

    area1 = (x1 - x0) * (y1 - y0)
    ltx = jnp.maximum(x0, X0)
    lty = jnp.maximum(y0, Y0)
    rbx = jnp.minimum(x1, X1)
    rby = jnp.minimum(y1, Y1)
    iw = jnp.clip(rbx - ltx, 0.0, None)
    ih = jnp.clip(rby - lty, 0.0, None)
    inter = iw * ih
    union = area1 + AREA2 - inter
    iou = inter / union
    ex = jnp.minimum(x0, X0)
    exr = jnp.maximum(x1, X1)
    ey = jnp.minimum(y0, Y0)
    eyb = jnp.maximum(y1, Y1)
    earea = jnp.clip(exr - ex, 0.0, None) * jnp.clip(eyb - ey, 0.0, None)
    giou = iou - (earea - union) / earea

    p = 1.0 / (1.0 + jnp.exp(-plog))
    one_m_p = 1.0 - p
    pos = ALPHA * one_m_p * one_m_p * (-jnp.log(p + 1e-8))
    neg = (1.0 - ALPHA) * p * p * (-jnp.log(1.0 - p + 1e-8))
    cost_class = pos - neg

    inv_w = jnp.float32(1.0) / jnp.float32(1333.0)
    inv_h = jnp.float32(1.0) / jnp.float32(800.0)
    cb = (jnp.abs(x0 * inv_w - TNX0) + jnp.abs(y0 * inv_h - TNY0)
          + jnp.abs(x1 * inv_w - TNX1) + jnp.abs(y1 * inv_h - TNY1))

    t0 = poses_ref[:, 0:1]
    t1 = poses_ref[:, 1:2]
    t2 = poses_ref[:, 2:3]
    r0 = poses_ref[:, 3:4]
    r1 = poses_ref[:, 4:5]
    r2 = poses_ref[:, 5:6]
    cpose = (jnp.abs(t0 - TT0) + jnp.abs(t1 - TT1) + jnp.abs(t2 - TT2)
             + jnp.abs(r0 - TR0) + jnp.abs(r1 - TR1) + jnp.abs(r2 - TR2))

    ax = (x0 + x1) * 0.5
    ay = (y0 + y1) * 0.5
    in_boxes = ((ax > BX0) & (ax < BX1) & (ay > BY0) & (ay < BY1))
    in_centers = ((ax > CLo) & (ax < CHi) & (ay > CTo) & (ay < CBo))
    both = in_boxes & in_centers
    fg = (jnp.sum(in_boxes.astype(jnp.float32), axis=1, keepdims=True) > 0.0) | \
         (jnp.sum(in_centers.astype(jnp.float32), axis=1, keepdims=True) > 0.0)

    cost = (COST_BBOX * cb + COST_CLASS * cost_class + COST_GIOU * (-giou)
            + 100.0 * (1.0 - both.astype(jnp.float32)) + cpose
            + 10000.0 * (1.0 - fg.astype(jnp.float32)))

    lane = lax.broadcasted_iota(jnp.int32, (BLK, L), 1)
    cost = jnp.where(lane < G, cost, BIG_F)
    iou = jnp.where(lane < G, iou, -1.0)
    return cost, iou


def _extract5_min(cur):
    ms = []
    for t in range(5):
        m = jnp.min(cur, axis=0, keepdims=True)
        ms.append(m)
        if t < 4:
            cur = jnp.where(cur <= m, SENT_F, cur)
    return jnp.concatenate(ms, axis=0)


def _extract5_max0(cur):
    ms = []
    for t in range(5):
        m = jnp.max(cur, axis=0, keepdims=True)
        ms.append(jnp.maximum(m, 0.0))
        if t < 4:
            cur = jnp.where(cur >= m, -SENT_F, cur)
    return jnp.concatenate(ms, axis=0)


def _threshold(s_cost, s_iou):
    s = (s_iou[0:1, :] + s_iou[1:2, :] + s_iou[2:3, :]
         + s_iou[3:4, :] + s_iou[4:5, :])
    t = s_cost[0:1, :]
    t = jnp.where(s >= 2.0, s_cost[1:2, :], t)
    t = jnp.where(s >= 3.0, s_cost[2:3, :], t)
    t = jnp.where(s >= 4.0, s_cost[3:4, :], t)
    t = jnp.where(s >= 5.0, s_cost[4:5, :], t)
    return t


def _ka(boxes_ref, poses_ref, lt_ref, oh_ref, pk1_ref, pk2_ref,
        sel_out, gt_out, qidx_out, reff_out,
        costS, s_cost, s_iou, acc_colsum, acc_bval, acc_bidx,
        acc_mval, acc_midx):
    pid = pl.program_id(0)

    @pl.when(pid == 0)
    def _init():
        s_cost[...] = jnp.full((8, L), SENT_F, jnp.float32)
        s_iou[...] = jnp.full((8, L), -SENT_F, jnp.float32)
        acc_colsum[...] = jnp.zeros((8, L), jnp.float32)
        acc_bval[...] = jnp.full((8, L), SENT_F, jnp.float32)
        acc_bidx[...] = jnp.zeros((8, L), jnp.int32)
        acc_mval[...] = jnp.full((8, L), SENT_F, jnp.float32)
        acc_midx[...] = jnp.zeros((8, L), jnp.int32)

    @pl.when(pid < NBD)
    def _phase0():
        d = lax.dot_general(lt_ref[...], oh_ref[...],
                            (((0,), (0,)), ((), ())),
                            precision=lax.Precision.HIGHEST,
                            preferred_element_type=jnp.float32)

        @pl.when(pid < NBD - 1)
        def _full():
            costS[pl.ds(pid * BLKD, BLKD), :] = d

        @pl.when(pid == NBD - 1)
        def _tail():
            costS[pl.ds((NBD - 1) * BLKD, N - (NBD - 1) * BLKD), :] = \
                d[0:N - (NBD - 1) * BLKD, :]

    @pl.when((pid >= NBD) & (pid < NBD + NB))
    def _phase1():
        b = pid - NBD
        cost, iou = _build_cost(boxes_ref, poses_ref,
                                costS[pl.ds(b * BLK, BLK), :],
                                pk1_ref, pk2_ref)
        costS[pl.ds(b * BLK, BLK), :] = cost
        blk5 = _extract5_min(cost)
        s_cost[0:5, :] = _extract5_min(
            jnp.concatenate([blk5, s_cost[0:5, :]], axis=0))
        blk5i = _extract5_max0(iou)
        s_iou[0:5, :] = _extract5_max0(
            jnp.concatenate([blk5i, s_iou[0:5, :]], axis=0))

    @pl.when((pid >= NBD + NB) & (pid < NBD + 2 * NB))
    def _phase2():
        b = pid - (NBD + NB)
        cost = costS[pl.ds(b * BLK, BLK), :]
        t = _threshold(s_cost, s_iou)

        lane = lax.broadcasted_iota(jnp.int32, (BLK, L), 1)
        valid = lane < G
        matching0 = (cost <= t) & valid
        amg = jnp.sum(matching0.astype(jnp.float32), axis=1, keepdims=True)

        rmin = jnp.min(cost, axis=1, keepdims=True)
        amin = jnp.min(jnp.where(cost == rmin, lane, BIG_I),
                       axis=1, keepdims=True)
        onehot_f = (lane == amin).astype(jnp.float32)
        m0f = matching0.astype(jnp.float32)
        conflict_f = (amg > 1.0).astype(jnp.float32)
        mf = conflict_f * onehot_f + (1.0 - conflict_f) * m0f

        matched = amg > 0.0
        matched_f = matched.astype(jnp.float32)
        firstlane = jnp.min(jnp.where(mf > 0.0, lane, BIG_I),
                            axis=1, keepdims=True)
        gt = jnp.where(matched, firstlane, 0)

        sel_out[...] = jnp.broadcast_to(matched_f, (BLK, 8))
        gt_out[...] = jnp.broadcast_to(gt, (BLK, 8))

        acc_colsum[0:1, :] = acc_colsum[0:1, :] + \
            jnp.sum(mf, axis=0, keepdims=True)

        rowg = lax.broadcasted_iota(jnp.int32, (BLK, L), 0) + b * BLK
        boosted = cost + 100000.0 * matched_f
        bval = jnp.min(boosted, axis=0, keepdims=True)
        bidx = jnp.min(jnp.where(boosted == bval, rowg, BIG_I),
                       axis=0, keepdims=True)
        old_v = acc_bval[0:1, :]
        old_i = acc_bidx[0:1, :]
        upd = bval < old_v
        acc_bval[0:1, :] = jnp.where(upd, bval, old_v)
        acc_bidx[0:1, :] = jnp.where(upd, bidx, old_i)

        mrow = jnp.where(matched, cost, SENT_F)
        mval = jnp.min(mrow, axis=0, keepdims=True)
        midx = jnp.min(jnp.where(mrow == mval, rowg, BIG_I),
                       axis=0, keepdims=True)
        old_v = acc_mval[0:1, :]
        old_i = acc_midx[0:1, :]
        upd = mval < old_v
        acc_mval[0:1, :] = jnp.where(upd, mval, old_v)
        acc_midx[0:1, :] = jnp.where(upd, midx, old_i)

    @pl.when(pid == NBD + 2 * NB)
    def _phase3():
        lane1 = lax.broadcasted_iota(jnp.int32, (1, L), 1)
        active = (acc_colsum[0:1, :] == 0.0) & (lane1 < G)
        r_row = jnp.where(active, acc_bidx[0:1, :], N)

        def body(i, carry):
            rv, ri = carry
            c = costS[pl.ds(i * BLK, BLK), :]
            rowg = lax.broadcasted_iota(jnp.int32, (BLK, L), 0) + i * BLK
            eq = rowg == r_row
            member = jnp.sum(eq.astype(jnp.float32), axis=1,
                             keepdims=True) > 0.0
            vals = jnp.where(member, c, SENT_F)
            v = jnp.min(vals, axis=0, keepdims=True)
            idx = jnp.min(jnp.where(vals == v, rowg, BIG_I),
                          axis=0, keepdims=True)
            upd = v < rv
            return (jnp.where(upd, v, rv), jnp.where(upd, idx, ri))

        rv0 = jnp.full((1, L), SENT_F, jnp.float32)
        ri0 = jnp.zeros((1, L), jnp.int32)
        rv, ri = lax.fori_loop(0, NB, body, (rv0, ri0))

        mv = acc_mval[0:1, :]
        mi = acc_midx[0:1, :]
        q = jnp.where(rv < mv, ri, mi)
        q = jnp.where(rv == mv, jnp.minimum(ri, mi), q)
        qidx_out[...] = jnp.broadcast_to(q, (8, L))

        io0 = lax.broadcasted_iota(jnp.int32, (L, L), 0)
        io1 = lax.broadcasted_iota(jnp.int32, (L, L), 1)
        ident = io0 == io1
        Rb = jnp.broadcast_to(r_row, (L, L))
        rT = jnp.min(jnp.where(ident, Rb, BIG_I), axis=1, keepdims=True)
        act_i = active.astype(jnp.int32)
        Ab = jnp.broadcast_to(act_i, (L, L))
        aT = jnp.min(jnp.where(ident, Ab, BIG_I), axis=1, keepdims=True)
        m2 = (rT == r_row) & (aT == 1)
        gval = jnp.min(jnp.where(m2, io0, BIG_I), axis=0, keepdims=True)

        idx_eff = jnp.where(active, 8 * acc_bidx[0:1, :], 1)
        reff_out[...] = jnp.concatenate(
            [r_row, idx_eff, gval,
             jnp.zeros((5, L), jnp.int32)], axis=0)


def _kb_sc(sel_in, gt_in, reff_in, sel_o, gt_o, idxv, gvalv, onesv,
           bufff, bufi, sem):
    c = lax.axis_index("c")
    s = lax.axis_index("s")
    wid = s * 2 + c
    ch = (N * 8) // 32
    base = wid * ch
    pltpu.sync_copy(sel_in.at[pl.ds(base, ch)], bufff)
    pltpu.sync_copy(bufff, sel_o.at[pl.ds(base, ch)])
    pltpu.sync_copy(gt_in.at[pl.ds(base, ch)], bufi)
    pltpu.sync_copy(bufi, gt_o.at[pl.ds(base, ch)])
    plsc.subcore_barrier()

    @pl.when(wid == 0)
    def _():
        pltpu.sync_copy(reff_in.at[1], idxv)
        pltpu.sync_copy(reff_in.at[2], gvalv)
        for k in range(8):
            onesv[pl.ds(k * 16, 16)] = jnp.full((16,), 1.0, jnp.float32)
        pltpu.async_copy(onesv, sel_o.at[idxv], sem).wait()
        pltpu.async_copy(gvalv, gt_o.at[idxv], sem).wait()


def _k3tc(sel0_ref, gt0_ref, reff_ref, sel_out, gtf_out):
    pid = pl.program_id(0)
    r_row = reff_ref[0:1, :]
    rowg = lax.broadcasted_iota(jnp.int32, (BLK, L), 0) + pid * BLK
    eq = rowg == r_row
    anyeq = jnp.sum(eq.astype(jnp.float32), axis=1, keepdims=True) > 0.0
    lane = lax.broadcasted_iota(jnp.int32, (BLK, L), 1)
    gmin = jnp.min(jnp.where(eq, lane, BIG_I), axis=1, keepdims=True)

    matched = sel0_ref[:, 0:1] > 0.0
    sel = matched | anyeq
    gt = jnp.where((~matched) & anyeq, gmin, gt0_ref[:, 0:1])
    sel_out[...] = jnp.broadcast_to(sel.astype(jnp.float32), (BLK, 8))
    gtf_out[...] = jnp.broadcast_to(gt, (BLK, 8))


def kernel(pred_logits, pred_boxes, pred_poses, tgt_labels, tgt_boxes,
           tgt_boxes_xyxy, P2s, image_size_xyxy, image_size_xyxy_tgt,
           translation_matrix, rotation_matrix, lwhs):
    boxes = pred_boxes[0]
    poses = pred_poses[0]
    logitsT = jnp.transpose(pred_logits[0])

    onehot = (tgt_labels[None, :] ==
              jnp.arange(C, dtype=tgt_labels.dtype)[:, None])
    onehot = jnp.pad(onehot.astype(jnp.float32), ((0, 0), (0, L - G)))

    tx0, ty0 = tgt_boxes_xyxy[:, 0], tgt_boxes_xyxy[:, 1]
    tx1, ty1 = tgt_boxes_xyxy[:, 2], tgt_boxes_xyxy[:, 3]
    tnorm = tgt_boxes_xyxy / image_size_xyxy_tgt
    area2 = (tx1 - tx0) * (ty1 - ty0)
    pk1 = jnp.stack([tx0, ty0, tx1, ty1,
                     tnorm[:, 0], tnorm[:, 1], tnorm[:, 2], tnorm[:, 3],
                     translation_matrix[:, 0], translation_matrix[:, 1],
                     translation_matrix[:, 2],
                     rotation_matrix[:, 0], rotation_matrix[:, 1],
                     rotation_matrix[:, 2], area2,
                     jnp.zeros_like(tx0)], axis=0)
    pk1 = jnp.pad(pk1, ((0, 0), (0, L - G)))

    tcx, tcy = (tx0 + tx1) * 0.5, (ty0 + ty1) * 0.5
    tw, th = tx1 - tx0, ty1 - ty0
    BX0, BY0 = tcx - 0.5 * tw, tcy - 0.5 * th
    BX1, BY1 = tcx + 0.5 * tw, tcy + 0.5 * th
    cr = 2.5
    CLo = tcx - cr * (BX1 - BX0)
    CHi = tcx + cr * (BX1 - BX0)
    CTo = tcy - cr * (BY1 - BY0)
    CBo = tcy + cr * (BY1 - BY0)
    pk2 = jnp.stack([BX0, BY0, BX1, BY1, CLo, CHi, CTo, CBo], axis=0)
    pk2 = jnp.pad(pk2, ((0, 0), (0, L - G)), constant_values=BIG_F)

    def in_map(i):
        return (jnp.clip(i - NBD, 0, NB - 1), 0)

    def out_map(i):
        return (jnp.clip(i - (NBD + NB), 0, NB - 1), 0)

    sel0, gt0, qidx, reff = _pallas_call(
        _ka,
        grid=(NBD + 2 * NB + 1,),
        in_specs=[
            pl.BlockSpec((BLK, 4), in_map),
            pl.BlockSpec((BLK, 6), in_map),
            pl.BlockSpec((C, BLKD), lambda i: (0, jnp.minimum(i, NBD - 1))),
            pl.BlockSpec((C, L), lambda i: (0, 0)),
            pl.BlockSpec((16, L), lambda i: (0, 0)),
            pl.BlockSpec((8, L), lambda i: (0, 0)),
        ],
        out_specs=[
            pl.BlockSpec((BLK, 8), out_map),
            pl.BlockSpec((BLK, 8), out_map),
            pl.BlockSpec((8, L), lambda i: (0, 0)),
            pl.BlockSpec((8, L), lambda i: (0, 0)),
        ],
        out_shape=[
            jax.ShapeDtypeStruct((N, 8), jnp.float32),
            jax.ShapeDtypeStruct((N, 8), jnp.int32),
            jax.ShapeDtypeStruct((8, L), jnp.int32),
            jax.ShapeDtypeStruct((8, L), jnp.int32),
        ],
        scratch_shapes=[
            pltpu.VMEM((N, L), jnp.float32),
            pltpu.VMEM((8, L), jnp.float32),
            pltpu.VMEM((8, L), jnp.float32),
            pltpu.VMEM((8, L), jnp.float32),
            pltpu.VMEM((8, L), jnp.float32),
            pltpu.VMEM((8, L), jnp.int32),
            pltpu.VMEM((8, L), jnp.float32),
            pltpu.VMEM((8, L), jnp.int32),
        ],
    )(boxes, poses, logitsT, onehot, pk1, pk2)

    kb = functools.partial(
        pl.kernel,
        mesh=plsc.VectorSubcoreMesh(core_axis_name="c", subcore_axis_name="s"),
        out_type=[
            jax.ShapeDtypeStruct((N * 8,), jnp.float32),
            jax.ShapeDtypeStruct((N * 8,), jnp.int32),
        ],
        scratch_types=[
            pltpu.VMEM((L,), jnp.int32),
            pltpu.VMEM((L,), jnp.int32),
            pltpu.VMEM((L,), jnp.float32),
            pltpu.VMEM(((N * 8) // 32,), jnp.float32),
            pltpu.VMEM(((N * 8) // 32,), jnp.int32),
            pltpu.SemaphoreType.DMA,
        ],
    )(_kb_sc)
    sel, gtf = kb(sel0.reshape(-1), gt0.reshape(-1), reff)

    selected_query = sel.reshape(N, 8)[:, 0] > 0.0
    gt_indices = gtf.reshape(N, 8)[:, 0]
    matched_query_id = qidx[0, :G]
    return selected_query, gt_indices, matched_query_id
